# baseline (speedup 1.0000x reference)
import functools

import jax
import jax.numpy as jnp
from jax import lax
from jax.experimental import pallas as pl
from jax.experimental.pallas import tpu as pltpu
from jax.experimental.pallas import tpu_sc as plsc


def _make_deg_kernel(epw, Np, NC, NS, K, nchunk):
    rpt = Np // NS
    mesh = plsc.VectorSubcoreMesh(core_axis_name="c", subcore_axis_name="s")

    @functools.partial(
        pl.kernel,
        out_type=jax.ShapeDtypeStruct((NC * Np,), jnp.float32),
        mesh=mesh,
        scratch_types=[
            pltpu.VMEM((nchunk, K), jnp.int32),
            pltpu.VMEM((K,), jnp.float32),
            pltpu.VMEM((rpt,), jnp.float32),
            pltpu.VMEM_SHARED((Np,), jnp.float32),
        ],
    )
    def deg_kernel(dst_hbm, out_hbm, didx, ones, stage, accd):
        cid = lax.axis_index("c")
        sid = lax.axis_index("s")
        wid = cid * NS + sid
        sl = pl.ds(sid * rpt, rpt)

        def zero_body(i, carry):
            stage[pl.ds(i * 16, 16)] = jnp.zeros((16,), jnp.float32)
            return carry

        lax.fori_loop(0, rpt // 16, zero_body, 0)
        pltpu.sync_copy(stage, accd.at[sl])
        pltpu.sync_copy(dst_hbm.at[wid], didx)
        for k in range(K // 16):
            ones[pl.ds(k * 16, 16)] = jnp.ones((16,), jnp.float32)
        plsc.subcore_barrier()

        def body(j, carry):
            pltpu.sync_copy(ones, accd.at[didx.at[j]], add=True)
            return carry

        lax.fori_loop(0, nchunk, body, 0)
        plsc.subcore_barrier()
        pltpu.sync_copy(accd.at[sl], stage)
        pltpu.sync_copy(stage, out_hbm.at[pl.ds(cid * Np + sid * rpt, rpt)])

    return deg_kernel


def _make_agg_kernel(N, Dh, NC, NS, K, nchunk, padr):
    assert nchunk % 2 == 0, nchunk
    Nacc = N + padr
    mesh = plsc.VectorSubcoreMesh(core_axis_name="c", subcore_axis_name="s")

    @functools.partial(
        pl.kernel,
        out_type=jax.ShapeDtypeStruct((NC, N, Dh), jnp.float32),
        mesh=mesh,
        scratch_types=[
            pltpu.VMEM((nchunk, K), jnp.int32),
            pltpu.VMEM((nchunk, K), jnp.int32),
            pltpu.VMEM((K, Dh), jnp.float32),
            pltpu.VMEM((K, Dh), jnp.float32),
            pltpu.VMEM_SHARED((Nacc, Dh), jnp.float32),
            pltpu.SemaphoreType.DMA,
            pltpu.SemaphoreType.DMA,
        ],
        compiler_params=pltpu.CompilerParams(use_tc_tiling_on_sc=False),
    )
    def agg_kernel(y_hbm, src_hbm, dst_hbm, zeros_hbm, out_hbm,
                   sidx, didx, rows_a, rows_b, acc, sem_a, sem_b):
        cid = lax.axis_index("c")
        sid = lax.axis_index("s")
        wid = cid * NS + sid
        R0 = 8 * (N // (8 * NS))
        rem = N - NS * R0

        def each_slice(fn):
            fn(pl.ds(sid * R0, R0))
            if rem:
                @pl.when(sid == NS - 1)
                def _():
                    fn(pl.ds(NS * R0, rem))

        @pl.when(cid == 0)
        def _():
            each_slice(lambda sl: pltpu.sync_copy(y_hbm.at[sl], acc.at[sl]))

        @pl.when(cid != 0)
        def _():
            each_slice(lambda sl: pltpu.sync_copy(zeros_hbm.at[sl], acc.at[sl]))

        pltpu.sync_copy(src_hbm.at[wid], sidx)
        pltpu.sync_copy(dst_hbm.at[wid], didx)
        plsc.subcore_barrier()

        pltpu.async_copy(y_hbm.at[sidx.at[0]], rows_a, sem_a)

        def body(g, carry):
            c0 = 2 * g
            pltpu.async_copy(y_hbm.at[sidx.at[c0 + 1]], rows_b, sem_b)
            pltpu.make_async_copy(y_hbm.at[sidx.at[c0]], rows_a, sem_a).wait()
            pltpu.sync_copy(rows_a, acc.at[didx.at[c0]], add=True)

            @pl.when(c0 + 2 < nchunk)
            def _():
                pltpu.async_copy(y_hbm.at[sidx.at[c0 + 2]], rows_a, sem_a)

            pltpu.make_async_copy(y_hbm.at[sidx.at[c0 + 1]], rows_b, sem_b).wait()
            pltpu.sync_copy(rows_b, acc.at[didx.at[c0 + 1]], add=True)
            return carry

        lax.fori_loop(0, nchunk // 2, body, 0)
        plsc.subcore_barrier()
        each_slice(lambda sl: pltpu.sync_copy(acc.at[sl], out_hbm.at[cid, sl]))

    return agg_kernel


def _dinv_of(d_ref):
    return lax.rsqrt(d_ref[:, 0:1] + d_ref[:, 1:2] + 1.0)


def _mm_scale(x, W, degT, blk):
    N, Dx = x.shape
    Dh = W.shape[1]

    def body(x_ref, w_ref, d_ref, o_ref):
        xw = jnp.dot(x_ref[...], w_ref[...], preferred_element_type=jnp.float32)
        o_ref[...] = xw * _dinv_of(d_ref)

    return pl.pallas_call(
        body,
        grid=(N // blk,),
        in_specs=[
            pl.BlockSpec((blk, Dx), lambda i: (i, 0)),
            pl.BlockSpec((Dx, Dh), lambda i: (0, 0)),
            pl.BlockSpec((blk, 2), lambda i: (i, 0)),
        ],
        out_specs=pl.BlockSpec((blk, Dh), lambda i: (i, 0)),
        out_shape=jax.ShapeDtypeStruct((N, Dh), jnp.float32),
    )(x, W, degT)


def _mid_layer(p, degT, b1, Wcat, blk):
    _, N, Dh = p.shape
    D2 = Wcat.shape[1]

    def body(p_ref, d_ref, b_ref, w_ref, o_ref):
        dinv = _dinv_of(d_ref)
        h = jnp.maximum(dinv * (p_ref[0] + p_ref[1]) + b_ref[...], 0.0)
        o_ref[...] = jnp.dot(h, w_ref[...], preferred_element_type=jnp.float32) * dinv

    return pl.pallas_call(
        body,
        grid=(N // blk,),
        in_specs=[
            pl.BlockSpec((2, blk, Dh), lambda i: (0, i, 0)),
            pl.BlockSpec((blk, 2), lambda i: (i, 0)),
            pl.BlockSpec((1, Dh), lambda i: (0, 0)),
            pl.BlockSpec((Dh, D2), lambda i: (0, 0)),
        ],
        out_specs=pl.BlockSpec((blk, D2), lambda i: (i, 0)),
        out_shape=jax.ShapeDtypeStruct((N, D2), jnp.float32),
    )(p, degT, b1, Wcat)


def _final_layer(q, degT, bcat, blk):
    _, N, D2 = q.shape

    def body(q_ref, d_ref, b_ref, o_ref):
        o_ref[...] = _dinv_of(d_ref) * (q_ref[0] + q_ref[1]) + b_ref[...]

    return pl.pallas_call(
        body,
        grid=(N // blk,),
        in_specs=[
            pl.BlockSpec((2, blk, D2), lambda i: (0, i, 0)),
            pl.BlockSpec((blk, 2), lambda i: (i, 0)),
            pl.BlockSpec((1, D2), lambda i: (0, 0)),
        ],
        out_specs=pl.BlockSpec((blk, D2), lambda i: (i, 0)),
        out_shape=jax.ShapeDtypeStruct((N, D2), jnp.float32),
    )(q, degT, bcat)


def kernel(x, edge_index, W1, b1, Wmu, bmu, Wlog, blog):
    N, D = x.shape
    E = edge_index.shape[1]
    try:
        info = plsc.get_sparse_core_info()
        NC, NS = info.num_cores, info.num_subcores
    except Exception:
        NC, NS = 2, 16
    NW = NC * NS

    K = 64
    epw = -(-E // NW)
    epw = ((epw + 2 * K - 1) // (2 * K)) * (2 * K)
    nchunk = epw // K
    Epad = NW * epw

    PADR = 256
    pad = Epad - E
    srcp = jnp.concatenate([edge_index[0],
                            jnp.zeros((pad,), edge_index.dtype)])
    dstp = jnp.concatenate([edge_index[1],
                            N + (jnp.arange(pad, dtype=edge_index.dtype) % PADR)])
    src3 = srcp.reshape(NW, nchunk, K)
    dst3 = dstp.reshape(NW, nchunk, K)

    Np = ((N + PADR + 16 * NS - 1) // (16 * NS)) * (16 * NS)

    deg_k = _make_deg_kernel(epw, Np, NC, NS, K, nchunk)
    degp = deg_k(dst3).reshape(NC, Np)
    degT = degp[:, :N].T

    zeros_nd = jnp.zeros((N, D), jnp.float32)
    blk = 2000 if N % 2000 == 0 else 8 * (N // 8)

    y1 = _mm_scale(x, W1, degT, blk)

    agg_k = _make_agg_kernel(N, W1.shape[1], NC, NS, K, nchunk, PADR)
    p = agg_k(y1, src3, dst3, zeros_nd)

    Wcat = jnp.concatenate([Wmu, Wlog], axis=1)
    bcat = jnp.concatenate([bmu, blog])[None, :]
    y2 = _mid_layer(p, degT, b1[None, :], Wcat, blk)

    q = agg_k(y2, src3, dst3, zeros_nd)

    out = _final_layer(q, degT, bcat, blk)
    L = Wmu.shape[1]
    return (out[:, :L], out[:, L:])


# speedup vs baseline: 1.7791x; 1.7791x over previous
import functools

import jax
import jax.numpy as jnp
from jax import lax
from jax.experimental import pallas as pl
from jax.experimental.pallas import tpu as pltpu
from jax.experimental.pallas import tpu_sc as plsc


def _pick_chunk(epw, maxk=128):
    for k in range(maxk, 7, -8):
        if epw % k == 0:
            return k
    raise ValueError(f"edges-per-tile {epw} has no valid chunk size")


def _make_deg_kernel(E, Np, NC, NS):
    NW = NC * NS
    epw = E // NW
    K = _pick_chunk(epw, maxk=80)
    nchunk = epw // K
    rpt = Np // NS
    mesh = plsc.VectorSubcoreMesh(core_axis_name="c", subcore_axis_name="s")

    @functools.partial(
        pl.kernel,
        out_type=jax.ShapeDtypeStruct((NC * Np,), jnp.float32),
        mesh=mesh,
        scratch_types=[
            pltpu.VMEM((nchunk, K), jnp.int32),
            pltpu.VMEM((K,), jnp.float32),
            pltpu.VMEM((rpt,), jnp.float32),
            pltpu.VMEM_SHARED((Np,), jnp.float32),
        ],
    )
    def deg_kernel(dst_hbm, out_hbm, didx, ones, stage, accd):
        cid = lax.axis_index("c")
        sid = lax.axis_index("s")
        wid = cid * NS + sid
        sl = pl.ds(sid * rpt, rpt)

        def zero_body(i, carry):
            stage[pl.ds(i * 16, 16)] = jnp.zeros((16,), jnp.float32)
            return carry

        lax.fori_loop(0, rpt // 16, zero_body, 0)
        pltpu.sync_copy(stage, accd.at[sl])
        pltpu.sync_copy(dst_hbm.at[wid], didx)
        for k in range(K // 16):
            ones[pl.ds(k * 16, 16)] = jnp.ones((16,), jnp.float32)
        plsc.subcore_barrier()

        def body(j, carry):
            pltpu.sync_copy(ones, accd.at[didx.at[j]], add=True)
            return carry

        lax.fori_loop(0, nchunk, body, 0)
        plsc.subcore_barrier()
        pltpu.sync_copy(accd.at[sl], stage)
        pltpu.sync_copy(stage, out_hbm.at[pl.ds(cid * Np + sid * rpt, rpt)])

    return deg_kernel


def _make_agg_kernel(N, Dh, E, NC, NS):
    NW = NC * NS
    epw = E // NW
    K = _pick_chunk(epw, maxk=80)
    nchunk = epw // K
    rpt = N // NS
    mesh = plsc.VectorSubcoreMesh(core_axis_name="c", subcore_axis_name="s")

    @functools.partial(
        pl.kernel,
        out_type=jax.ShapeDtypeStruct((NC, N, Dh), jnp.float32),
        mesh=mesh,
        scratch_types=[
            pltpu.VMEM((nchunk, K), jnp.int32),
            pltpu.VMEM((nchunk, K), jnp.int32),
            pltpu.VMEM((K, Dh), jnp.float32),
            pltpu.VMEM((K, Dh), jnp.float32),
            pltpu.VMEM_SHARED((N, Dh), jnp.float32),
            pltpu.SemaphoreType.DMA,
            pltpu.SemaphoreType.DMA,
        ],
        compiler_params=pltpu.CompilerParams(use_tc_tiling_on_sc=False),
    )
    def agg_kernel(y_hbm, src_hbm, dst_hbm, zeros_hbm, out_hbm,
                   sidx, didx, rows_a, rows_b, acc, sem_a, sem_b):
        cid = lax.axis_index("c")
        sid = lax.axis_index("s")
        wid = cid * NS + sid
        R0 = 8 * (N // (8 * NS))
        rem = N - NS * R0

        def each_slice(fn):
            fn(pl.ds(sid * R0, R0))
            if rem:
                @pl.when(sid == NS - 1)
                def _():
                    fn(pl.ds(NS * R0, rem))

        @pl.when(cid == 0)
        def _():
            each_slice(lambda sl: pltpu.sync_copy(y_hbm.at[sl], acc.at[sl]))

        @pl.when(cid != 0)
        def _():
            each_slice(lambda sl: pltpu.sync_copy(zeros_hbm.at[sl], acc.at[sl]))

        pltpu.sync_copy(src_hbm.at[wid], sidx)
        pltpu.sync_copy(dst_hbm.at[wid], didx)
        plsc.subcore_barrier()

        half = nchunk // 2
        pltpu.async_copy(y_hbm.at[sidx.at[0]], rows_a, sem_a)

        def body(g, carry):
            c0 = 2 * g
            pltpu.async_copy(y_hbm.at[sidx.at[c0 + 1]], rows_b, sem_b)
            pltpu.make_async_copy(y_hbm.at[sidx.at[c0]], rows_a, sem_a).wait()
            pltpu.sync_copy(rows_a, acc.at[didx.at[c0]], add=True)
            if nchunk % 2:
                pltpu.async_copy(y_hbm.at[sidx.at[c0 + 2]], rows_a, sem_a)
            else:
                @pl.when(c0 + 2 < nchunk)
                def _():
                    pltpu.async_copy(y_hbm.at[sidx.at[c0 + 2]], rows_a, sem_a)
            pltpu.make_async_copy(y_hbm.at[sidx.at[c0 + 1]], rows_b, sem_b).wait()
            pltpu.sync_copy(rows_b, acc.at[didx.at[c0 + 1]], add=True)
            return carry

        lax.fori_loop(0, half, body, 0)
        if nchunk % 2:
            j = nchunk - 1
            pltpu.make_async_copy(y_hbm.at[sidx.at[j]], rows_a, sem_a).wait()
            pltpu.sync_copy(rows_a, acc.at[didx.at[j]], add=True)
        plsc.subcore_barrier()
        each_slice(lambda sl: pltpu.sync_copy(acc.at[sl], out_hbm.at[cid, sl]))

    return agg_kernel


def _dinv_of(d_ref):
    return lax.rsqrt(d_ref[:, 0:1] + d_ref[:, 1:2] + 1.0)


def _mm_scale(x, W, degT, blk):
    N, Dx = x.shape
    Dh = W.shape[1]

    def body(x_ref, w_ref, d_ref, o_ref):
        xw = jnp.dot(x_ref[...], w_ref[...], preferred_element_type=jnp.float32)
        o_ref[...] = xw * _dinv_of(d_ref)

    return pl.pallas_call(
        body,
        grid=(N // blk,),
        in_specs=[
            pl.BlockSpec((blk, Dx), lambda i: (i, 0)),
            pl.BlockSpec((Dx, Dh), lambda i: (0, 0)),
            pl.BlockSpec((blk, 2), lambda i: (i, 0)),
        ],
        out_specs=pl.BlockSpec((blk, Dh), lambda i: (i, 0)),
        out_shape=jax.ShapeDtypeStruct((N, Dh), jnp.float32),
    )(x, W, degT)


def _mid_layer(p, degT, b1, Wcat, blk):
    _, N, Dh = p.shape
    D2 = Wcat.shape[1]

    def body(p_ref, d_ref, b_ref, w_ref, o_ref):
        dinv = _dinv_of(d_ref)
        h = jnp.maximum(dinv * (p_ref[0] + p_ref[1]) + b_ref[...], 0.0)
        o_ref[...] = jnp.dot(h, w_ref[...], preferred_element_type=jnp.float32) * dinv

    return pl.pallas_call(
        body,
        grid=(N // blk,),
        in_specs=[
            pl.BlockSpec((2, blk, Dh), lambda i: (0, i, 0)),
            pl.BlockSpec((blk, 2), lambda i: (i, 0)),
            pl.BlockSpec((1, Dh), lambda i: (0, 0)),
            pl.BlockSpec((Dh, D2), lambda i: (0, 0)),
        ],
        out_specs=pl.BlockSpec((blk, D2), lambda i: (i, 0)),
        out_shape=jax.ShapeDtypeStruct((N, D2), jnp.float32),
    )(p, degT, b1, Wcat)


def _final_layer(q, degT, bcat, blk):
    _, N, D2 = q.shape

    def body(q_ref, d_ref, b_ref, o_ref):
        o_ref[...] = _dinv_of(d_ref) * (q_ref[0] + q_ref[1]) + b_ref[...]

    return pl.pallas_call(
        body,
        grid=(N // blk,),
        in_specs=[
            pl.BlockSpec((2, blk, D2), lambda i: (0, i, 0)),
            pl.BlockSpec((blk, 2), lambda i: (i, 0)),
            pl.BlockSpec((1, D2), lambda i: (0, 0)),
        ],
        out_specs=pl.BlockSpec((blk, D2), lambda i: (i, 0)),
        out_shape=jax.ShapeDtypeStruct((N, D2), jnp.float32),
    )(q, degT, bcat)


def kernel(x, edge_index, W1, b1, Wmu, bmu, Wlog, blog):
    N, D = x.shape
    E = edge_index.shape[1]
    try:
        info = plsc.get_sparse_core_info()
        NC, NS = info.num_cores, info.num_subcores
    except Exception:
        NC, NS = 2, 16
    NW = NC * NS
    assert E % NW == 0, (E, NW)
    epw = E // NW
    K = _pick_chunk(epw, maxk=80)
    nchunk = epw // K

    Np = ((N + 16 * NS - 1) // (16 * NS)) * (16 * NS)

    src3 = edge_index[0].reshape(NW, nchunk, K)
    dst3 = edge_index[1].reshape(NW, nchunk, K)

    deg_k = _make_deg_kernel(E, Np, NC, NS)
    degp = deg_k(dst3).reshape(NC, Np)
    degT = degp[:, :N].T

    zeros_nd = jnp.zeros((N, D), jnp.float32)
    blk = 2000 if N % 2000 == 0 else 8 * (N // 8)

    y1 = _mm_scale(x, W1, degT, blk)

    agg_k = _make_agg_kernel(N, W1.shape[1], E, NC, NS)
    p = agg_k(y1, src3, dst3, zeros_nd)

    Wcat = jnp.concatenate([Wmu, Wlog], axis=1)
    bcat = jnp.concatenate([bmu, blog])[None, :]
    y2 = _mid_layer(p, degT, b1[None, :], Wcat, blk)

    q = agg_k(y2, src3, dst3, zeros_nd)

    out = _final_layer(q, degT, bcat, blk)
    L = Wmu.shape[1]
    return (out[:, :L], out[:, L:])


# speedup vs baseline: 1.8039x; 1.0139x over previous
import functools

import jax
import jax.numpy as jnp
from jax import lax
from jax.experimental import pallas as pl
from jax.experimental.pallas import tpu as pltpu
from jax.experimental.pallas import tpu_sc as plsc


def _pick_chunk(epw, maxk=128):
    for k in range(maxk, 7, -8):
        if epw % k == 0:
            return k
    raise ValueError(f"edges-per-tile {epw} has no valid chunk size")


def _make_deg_kernel(E, Np, NC, NS):
    NW = NC * NS
    epw = E // NW
    K = _pick_chunk(epw, maxk=80)
    nchunk = epw // K
    rpt = Np // NS
    mesh = plsc.VectorSubcoreMesh(core_axis_name="c", subcore_axis_name="s")

    @functools.partial(
        pl.kernel,
        out_type=jax.ShapeDtypeStruct((NC * Np,), jnp.float32),
        mesh=mesh,
        scratch_types=[
            pltpu.VMEM((nchunk, K), jnp.int32),
            pltpu.VMEM((K,), jnp.float32),
            pltpu.VMEM((rpt,), jnp.float32),
            pltpu.VMEM_SHARED((Np,), jnp.float32),
        ],
    )
    def deg_kernel(dst_hbm, out_hbm, didx, ones, stage, accd):
        cid = lax.axis_index("c")
        sid = lax.axis_index("s")
        wid = cid * NS + sid
        sl = pl.ds(sid * rpt, rpt)

        def zero_body(i, carry):
            stage[pl.ds(i * 16, 16)] = jnp.zeros((16,), jnp.float32)
            return carry

        lax.fori_loop(0, rpt // 16, zero_body, 0)
        pltpu.sync_copy(stage, accd.at[sl])
        pltpu.sync_copy(dst_hbm.at[wid], didx)
        for k in range(K // 16):
            ones[pl.ds(k * 16, 16)] = jnp.ones((16,), jnp.float32)
        plsc.subcore_barrier()

        def body(j, carry):
            pltpu.sync_copy(ones, accd.at[didx.at[j]], add=True)
            return carry

        lax.fori_loop(0, nchunk, body, 0)
        plsc.subcore_barrier()
        pltpu.sync_copy(accd.at[sl], stage)
        pltpu.sync_copy(stage, out_hbm.at[pl.ds(cid * Np + sid * rpt, rpt)])

    return deg_kernel


def _make_agg_kernel(N, Dh, E, NC, NS):
    NW = NC * NS
    epw = E // NW
    K = _pick_chunk(epw, maxk=80)
    nchunk = epw // K
    rpt = N // NS
    mesh = plsc.VectorSubcoreMesh(core_axis_name="c", subcore_axis_name="s")

    @functools.partial(
        pl.kernel,
        out_type=jax.ShapeDtypeStruct((NC, N, Dh), jnp.float32),
        mesh=mesh,
        scratch_types=[
            pltpu.VMEM((nchunk, K), jnp.int32),
            pltpu.VMEM((nchunk, K), jnp.int32),
            pltpu.VMEM((K, Dh), jnp.float32),
            pltpu.VMEM((K, Dh), jnp.float32),
            pltpu.VMEM_SHARED((N, Dh), jnp.float32),
            pltpu.SemaphoreType.DMA,
            pltpu.SemaphoreType.DMA,
        ],
        compiler_params=pltpu.CompilerParams(use_tc_tiling_on_sc=False),
    )
    def agg_kernel(y_hbm, src_hbm, dst_hbm, zeros_hbm, out_hbm,
                   sidx, didx, rows_a, rows_b, acc, sem_a, sem_b):
        cid = lax.axis_index("c")
        sid = lax.axis_index("s")
        wid = cid * NS + sid
        R0 = 8 * (N // (8 * NS))
        rem = N - NS * R0

        def each_slice(fn):
            fn(pl.ds(sid * R0, R0))
            if rem:
                @pl.when(sid == NS - 1)
                def _():
                    fn(pl.ds(NS * R0, rem))

        @pl.when(cid == 0)
        def _():
            each_slice(lambda sl: pltpu.sync_copy(y_hbm.at[sl], acc.at[sl]))

        @pl.when(cid != 0)
        def _():
            each_slice(lambda sl: pltpu.sync_copy(zeros_hbm.at[sl], acc.at[sl]))

        pltpu.sync_copy(src_hbm.at[wid], sidx)
        pltpu.sync_copy(dst_hbm.at[wid], didx)
        plsc.subcore_barrier()

        half = nchunk // 2
        pltpu.async_copy(y_hbm.at[sidx.at[0]], rows_a, sem_a)

        def body(g, carry):
            c0 = 2 * g
            pltpu.async_copy(y_hbm.at[sidx.at[c0 + 1]], rows_b, sem_b)
            pltpu.make_async_copy(y_hbm.at[sidx.at[c0]], rows_a, sem_a).wait()
            pltpu.sync_copy(rows_a, acc.at[didx.at[c0]], add=True)
            if nchunk % 2:
                pltpu.async_copy(y_hbm.at[sidx.at[c0 + 2]], rows_a, sem_a)
            else:
                @pl.when(c0 + 2 < nchunk)
                def _():
                    pltpu.async_copy(y_hbm.at[sidx.at[c0 + 2]], rows_a, sem_a)
            pltpu.make_async_copy(y_hbm.at[sidx.at[c0 + 1]], rows_b, sem_b).wait()
            pltpu.sync_copy(rows_b, acc.at[didx.at[c0 + 1]], add=True)
            return carry

        lax.fori_loop(0, half, body, 0)
        if nchunk % 2:
            j = nchunk - 1
            pltpu.make_async_copy(y_hbm.at[sidx.at[j]], rows_a, sem_a).wait()
            pltpu.sync_copy(rows_a, acc.at[didx.at[j]], add=True)
        plsc.subcore_barrier()
        each_slice(lambda sl: pltpu.sync_copy(acc.at[sl], out_hbm.at[cid, sl]))

    return agg_kernel


def _dinv_of(d_ref):
    return lax.rsqrt(d_ref[:, 0:1] + d_ref[:, 1:2] + 1.0)


def _mm_scale(x, W, degT, blk):
    N, Dx = x.shape
    Dh = W.shape[1]

    def body(x_ref, w_ref, d_ref, o_ref):
        xw = jnp.dot(x_ref[...], w_ref[...], preferred_element_type=jnp.float32)
        o_ref[...] = xw * _dinv_of(d_ref)

    return pl.pallas_call(
        body,
        grid=(N // blk,),
        in_specs=[
            pl.BlockSpec((blk, Dx), lambda i: (i, 0)),
            pl.BlockSpec((Dx, Dh), lambda i: (0, 0)),
            pl.BlockSpec((blk, 2), lambda i: (i, 0)),
        ],
        out_specs=pl.BlockSpec((blk, Dh), lambda i: (i, 0)),
        out_shape=jax.ShapeDtypeStruct((N, Dh), jnp.float32),
    )(x, W, degT)


def _mid_layer(p, degT, b1, Wcat, blk):
    _, N, Dh = p.shape
    D2 = Wcat.shape[1]

    def body(p_ref, d_ref, b_ref, w_ref, o_ref):
        dinv = _dinv_of(d_ref)
        h = jnp.maximum(dinv * (p_ref[0] + p_ref[1]) + b_ref[...], 0.0)
        o_ref[...] = jnp.dot(h, w_ref[...], preferred_element_type=jnp.float32) * dinv

    return pl.pallas_call(
        body,
        grid=(N // blk,),
        in_specs=[
            pl.BlockSpec((2, blk, Dh), lambda i: (0, i, 0)),
            pl.BlockSpec((blk, 2), lambda i: (i, 0)),
            pl.BlockSpec((1, Dh), lambda i: (0, 0)),
            pl.BlockSpec((Dh, D2), lambda i: (0, 0)),
        ],
        out_specs=pl.BlockSpec((blk, D2), lambda i: (i, 0)),
        out_shape=jax.ShapeDtypeStruct((N, D2), jnp.float32),
    )(p, degT, b1, Wcat)


def _final_layer(q, degT, bmu, blog, blk, L):
    _, N, D2 = q.shape

    def body(q_ref, d_ref, bm_ref, bl_ref, mu_ref, lg_ref):
        s = _dinv_of(d_ref) * (q_ref[0] + q_ref[1])
        mu_ref[...] = s[:, :L] + bm_ref[...]
        lg_ref[...] = s[:, L:] + bl_ref[...]

    return pl.pallas_call(
        body,
        grid=(N // blk,),
        in_specs=[
            pl.BlockSpec((2, blk, D2), lambda i: (0, i, 0)),
            pl.BlockSpec((blk, 2), lambda i: (i, 0)),
            pl.BlockSpec((1, L), lambda i: (0, 0)),
            pl.BlockSpec((1, L), lambda i: (0, 0)),
        ],
        out_specs=[
            pl.BlockSpec((blk, L), lambda i: (i, 0)),
            pl.BlockSpec((blk, L), lambda i: (i, 0)),
        ],
        out_shape=[
            jax.ShapeDtypeStruct((N, L), jnp.float32),
            jax.ShapeDtypeStruct((N, L), jnp.float32),
        ],
    )(q, degT, bmu[None, :], blog[None, :])


def kernel(x, edge_index, W1, b1, Wmu, bmu, Wlog, blog):
    N, D = x.shape
    E = edge_index.shape[1]
    try:
        info = plsc.get_sparse_core_info()
        NC, NS = info.num_cores, info.num_subcores
    except Exception:
        NC, NS = 2, 16
    NW = NC * NS
    assert E % NW == 0, (E, NW)
    epw = E // NW
    K = _pick_chunk(epw, maxk=80)
    nchunk = epw // K

    Np = ((N + 16 * NS - 1) // (16 * NS)) * (16 * NS)

    src3 = edge_index[0].reshape(NW, nchunk, K)
    dst3 = edge_index[1].reshape(NW, nchunk, K)

    deg_k = _make_deg_kernel(E, Np, NC, NS)
    degp = deg_k(dst3).reshape(NC, Np)
    degT = degp[:, :N].T

    zeros_nd = jnp.zeros((N, D), jnp.float32)
    blk = 2000 if N % 2000 == 0 else 8 * (N // 8)

    y1 = _mm_scale(x, W1, degT, blk)

    agg_k = _make_agg_kernel(N, W1.shape[1], E, NC, NS)
    p = agg_k(y1, src3, dst3, zeros_nd)

    Wcat = jnp.concatenate([Wmu, Wlog], axis=1)
    y2 = _mid_layer(p, degT, b1[None, :], Wcat, blk)

    q = agg_k(y2, src3, dst3, zeros_nd)

    mu, log = _final_layer(q, degT, bmu, blog, blk, Wmu.shape[1])
    return (mu, log)
